# baseline (speedup 1.0000x reference)
import functools

import jax
import jax.numpy as jnp
from jax import lax
from jax.experimental import pallas as pl
from jax.experimental.pallas import tpu as pltpu
from jax.experimental.pallas import tpu_sc as plsc

N = 10000
E = 320000
D = 128
H = 32

NC = 2
NS = 16
NW = NC * NS
EW = E // NW
IB = 250
NB = EW // IB
NBUF = 8
RPT = N // NS
R = 2000
GRID = N // R

_mesh = plsc.VectorSubcoreMesh(core_axis_name="c", subcore_axis_name="s")


def _deg_body(dst_hbm, out_hbm, idx_v, hist_v):
    c = lax.axis_index("c")
    s = lax.axis_index("s")
    wid = s * NC + c
    ones = jnp.full((16,), 1.0, jnp.float32)
    zeros = jnp.zeros((16,), jnp.float32)

    def zero_chunk(gi):
        @plsc.parallel_loop(0, R // 16, unroll=8)
        def _(i):
            hist_v[gi, pl.ds(i * 16, 16)] = zeros

    for t in range(2):
        for gi in range(GRID):
            zero_chunk(gi)
        pltpu.sync_copy(dst_hbm.at[t, wid], idx_v)

        @plsc.parallel_loop(0, EW // 16, unroll=8)
        def _(i):
            iv = idx_v[pl.ds(i * 16, 16)]
            row = (
                (iv >= R).astype(jnp.int32)
                + (iv >= 2 * R).astype(jnp.int32)
                + (iv >= 3 * R).astype(jnp.int32)
                + (iv >= 4 * R).astype(jnp.int32)
            )
            col = iv - row * R
            plsc.addupdate_scatter(hist_v, [row, col], ones)

        for gi in range(GRID):
            pltpu.sync_copy(hist_v.at[pl.ds(gi, 1)],
                            out_hbm.at[t, gi, pl.ds(wid, 1)])


_deg_call = pl.kernel(
    _deg_body,
    out_type=jax.ShapeDtypeStruct((2, GRID, NW, R), jnp.float32),
    mesh=_mesh,
    compiler_params=pltpu.CompilerParams(needs_layout_passes=False),
    scratch_types=[
        pltpu.VMEM((EW,), jnp.int32),
        pltpu.VMEM((GRID, R), jnp.float32),
    ],
)


def _agg_body(gn_hbm, gr_hbm, src_hbm, dst_hbm, sp_hbm,
              an_sh, ar_sh, src_v, dst_v, rows_v, gsem, ssem):
    c = lax.axis_index("c")
    s = lax.axis_index("s")
    wid = s * NC + c
    r0 = s * 1000

    @pl.when(s < 10)
    def _():
        pltpu.sync_copy(gn_hbm.at[pl.ds(r0, 1000)], an_sh.at[pl.ds(r0, 1000)])
        pltpu.sync_copy(gr_hbm.at[pl.ds(r0, 1000)], ar_sh.at[pl.ds(r0, 1000)])
    plsc.subcore_barrier()
    for t, (g_hbm, a_sh) in enumerate(((gn_hbm, an_sh), (gr_hbm, ar_sh))):
        pltpu.sync_copy(src_hbm.at[t, wid], src_v)
        pltpu.sync_copy(dst_hbm.at[t, wid], dst_v)

        for b in range(NBUF // 2):
            pltpu.async_copy(g_hbm.at[src_v.at[b]], rows_v.at[b], gsem.at[b])

        def step(j, b, bn):
            pltpu.make_async_copy(
                g_hbm.at[src_v.at[j]], rows_v.at[b], gsem.at[b]).wait()
            pltpu.async_copy(rows_v.at[b], a_sh.at[dst_v.at[j]], ssem.at[b],
                             add=True)
            jn = j + NBUF // 2

            @pl.when(jn < NB)
            def _():
                @pl.when(jn >= NBUF)
                def _():
                    pltpu.make_async_copy(
                        rows_v.at[bn], a_sh.at[dst_v.at[jn - NBUF]],
                        ssem.at[bn]).wait()
                pltpu.async_copy(
                    g_hbm.at[src_v.at[jn]], rows_v.at[bn], gsem.at[bn])

        def outer(o, carry):
            for b in range(NBUF):
                j = o * NBUF + b
                step(j, b, (b + NBUF // 2) % NBUF)
            return carry
        lax.fori_loop(0, NB // NBUF, outer, 0)

        for b in range(NBUF):
            j_last = NB - NBUF + b
            pltpu.make_async_copy(
                rows_v.at[b], a_sh.at[dst_v.at[j_last]], ssem.at[b]).wait()
    plsc.subcore_barrier()

    @pl.when(s < 10)
    def _():
        for t, a_sh in enumerate((an_sh, ar_sh)):
            pltpu.sync_copy(a_sh.at[pl.ds(r0, 1000)],
                            sp_hbm.at[c, t, pl.ds(r0, 1000)])


_agg_call = pl.kernel(
    _agg_body,
    out_type=jax.ShapeDtypeStruct((NC, 2, N, H), jnp.float32),
    mesh=_mesh,
    compiler_params=pltpu.CompilerParams(use_tc_tiling_on_sc=False),
    scratch_types=[
        pltpu.VMEM_SHARED((N, H), jnp.float32),
        pltpu.VMEM_SHARED((N, H), jnp.float32),
        pltpu.VMEM((NB, IB), jnp.int32),
        pltpu.VMEM((NB, IB), jnp.int32),
        pltpu.VMEM((NBUF, IB, H), jnp.float32),
        pltpu.SemaphoreType.DMA((NBUF,)),
        pltpu.SemaphoreType.DMA((NBUF,)),
    ],
)


def _mm_body(x_ref, w2_ref, hn_ref, hr_ref):
    h = jnp.dot(x_ref[...], w2_ref[...], preferred_element_type=jnp.float32)
    hn_ref[...] = h[:, :H]
    hr_ref[...] = h[:, H:]


def _mm_call(x, w2):
    return pl.pallas_call(
        _mm_body,
        grid=(GRID,),
        in_specs=[
            pl.BlockSpec((R, D), lambda i: (i, 0)),
            pl.BlockSpec((D, 2 * H), lambda i: (0, 0)),
        ],
        out_specs=[
            pl.BlockSpec((R, H), lambda i: (i, 0)),
            pl.BlockSpec((R, H), lambda i: (i, 0)),
        ],
        out_shape=[
            jax.ShapeDtypeStruct((N, H), jnp.float32),
            jax.ShapeDtypeStruct((N, H), jnp.float32),
        ],
    )(x, w2)


def _scale_body(hn_ref, hr_ref, hist_ref, gn_ref, gr_ref):
    deg = 1.0 + jnp.sum(hist_ref[...], axis=2)[:, 0, :]
    dinv = lax.rsqrt(deg)
    gn_ref[...] = hn_ref[...] * dinv[0][:, None]
    gr_ref[...] = hr_ref[...] * dinv[1][:, None]


def _scale_call(hn, hr, hist):
    return pl.pallas_call(
        _scale_body,
        grid=(GRID,),
        in_specs=[
            pl.BlockSpec((R, H), lambda i: (i, 0)),
            pl.BlockSpec((R, H), lambda i: (i, 0)),
            pl.BlockSpec((2, 1, NW, R), lambda i: (0, i, 0, 0)),
        ],
        out_specs=[
            pl.BlockSpec((R, H), lambda i: (i, 0)),
            pl.BlockSpec((R, H), lambda i: (i, 0)),
        ],
        out_shape=[
            jax.ShapeDtypeStruct((N, H), jnp.float32),
            jax.ShapeDtypeStruct((N, H), jnp.float32),
        ],
    )(hn, hr, hist)


def _fin_body(sp_ref, gn_ref, gr_ref, hist_ref, bb_ref, lw_ref, lb_ref, out_ref):
    deg = 1.0 + jnp.sum(hist_ref[...], axis=2)[:, 0, :]
    dinv = lax.rsqrt(deg)
    sp = sp_ref[...]
    yn = (sp[0, 0] + sp[1, 0] - gn_ref[...]) * dinv[0][:, None]
    yr = (sp[0, 1] + sp[1, 1] - gr_ref[...]) * dinv[1][:, None]
    y = yn + yr + (bb_ref[0] + bb_ref[1])[None, :]
    r = jnp.maximum(y, 0.0)
    out_ref[...] = (
        jnp.dot(r, lw_ref[...], preferred_element_type=jnp.float32) + lb_ref[0, 0]
    )


def _fin_call(sp, gn, gr, hist, bb, lw, lb):
    return pl.pallas_call(
        _fin_body,
        grid=(GRID,),
        in_specs=[
            pl.BlockSpec((NC, 2, R, H), lambda i: (0, 0, i, 0)),
            pl.BlockSpec((R, H), lambda i: (i, 0)),
            pl.BlockSpec((R, H), lambda i: (i, 0)),
            pl.BlockSpec((2, 1, NW, R), lambda i: (0, i, 0, 0)),
            pl.BlockSpec((2, H), lambda i: (0, 0)),
            pl.BlockSpec((H, 1), lambda i: (0, 0)),
            pl.BlockSpec((1, 1), lambda i: (0, 0)),
        ],
        out_specs=pl.BlockSpec((R, 1), lambda i: (i, 0)),
        out_shape=jax.ShapeDtypeStruct((N, 1), jnp.float32),
    )(sp, gn, gr, hist, bb, lw, lb)


def kernel(x_house, edge_index_near, edge_index_road,
           W_near, b_near, W_road, b_road, lin_W, lin_b):
    dst2 = jnp.stack([edge_index_near[1], edge_index_road[1]]).reshape(2, NW, EW)
    w2 = jnp.concatenate([W_near, W_road], axis=1)
    hn, hr = _mm_call(x_house, w2)
    hist = _deg_call(dst2)
    gn, gr = _scale_call(hn, hr, hist)
    src4 = jnp.stack([edge_index_near[0], edge_index_road[0]]).reshape(2, NW, NB, IB)
    dst4 = dst2.reshape(2, NW, NB, IB)
    sp = _agg_call(gn, gr, src4, dst4)
    bb = jnp.stack([b_near, b_road])
    out = _fin_call(sp, gn, gr, hist, bb, lin_W, lin_b.reshape(1, 1))
    return out[:, 0]


# speedup vs baseline: 1.0605x; 1.0605x over previous
import functools

import jax
import jax.numpy as jnp
from jax import lax
from jax.experimental import pallas as pl
from jax.experimental.pallas import tpu as pltpu
from jax.experimental.pallas import tpu_sc as plsc

N = 10000
E = 320000
D = 128
H = 32

NC = 2
NS = 16
NW = NC * NS
EW = E // NW
IB = 250
NB = EW // IB
NBUF = 8
RPT = N // NS
R = 2000
GRID = N // R

_mesh = plsc.VectorSubcoreMesh(core_axis_name="c", subcore_axis_name="s")


def _deg_body(dst_hbm, out_hbm, idx_v, hist_v):
    c = lax.axis_index("c")
    s = lax.axis_index("s")
    wid = s * NC + c
    ones = jnp.full((16,), 1.0, jnp.float32)
    zeros = jnp.zeros((16,), jnp.float32)

    def zero_chunk(gi):
        @plsc.parallel_loop(0, R // 16, unroll=8)
        def _(i):
            hist_v[gi, pl.ds(i * 16, 16)] = zeros

    for t in range(2):
        for gi in range(GRID):
            zero_chunk(gi)
        pltpu.sync_copy(dst_hbm.at[t, wid], idx_v)

        @plsc.parallel_loop(0, EW // 16, unroll=8)
        def _(i):
            iv = idx_v[pl.ds(i * 16, 16)]
            row = (
                (iv >= R).astype(jnp.int32)
                + (iv >= 2 * R).astype(jnp.int32)
                + (iv >= 3 * R).astype(jnp.int32)
                + (iv >= 4 * R).astype(jnp.int32)
            )
            col = iv - row * R
            plsc.addupdate_scatter(hist_v, [row, col], ones)

        for gi in range(GRID):
            pltpu.sync_copy(hist_v.at[pl.ds(gi, 1)],
                            out_hbm.at[t, gi, pl.ds(wid, 1)])


_deg_call = pl.kernel(
    _deg_body,
    out_type=jax.ShapeDtypeStruct((2, GRID, NW, R), jnp.float32),
    mesh=_mesh,
    compiler_params=pltpu.CompilerParams(needs_layout_passes=False),
    scratch_types=[
        pltpu.VMEM((EW,), jnp.int32),
        pltpu.VMEM((GRID, R), jnp.float32),
    ],
)


def _agg_body(gn_hbm, gr_hbm, src_hbm, dst_hbm, sp_hbm,
              an_sh, ar_sh, src_v, dst_v, rows_v, gsem, ssem):
    c = lax.axis_index("c")
    s = lax.axis_index("s")
    wid = s * NC + c
    r0 = s * 1000

    @pl.when(s < 10)
    def _():
        pltpu.sync_copy(gn_hbm.at[pl.ds(r0, 1000)], an_sh.at[pl.ds(r0, 1000)])
        pltpu.sync_copy(gr_hbm.at[pl.ds(r0, 1000)], ar_sh.at[pl.ds(r0, 1000)])
    plsc.subcore_barrier()
    for t, (g_hbm, a_sh) in enumerate(((gn_hbm, an_sh), (gr_hbm, ar_sh))):
        pltpu.sync_copy(src_hbm.at[t, wid], src_v)
        pltpu.sync_copy(dst_hbm.at[t, wid], dst_v)

        for b in range(NBUF // 2):
            pltpu.async_copy(g_hbm.at[src_v.at[b]], rows_v.at[b], gsem.at[b])

        def step(j, b, bn):
            pltpu.make_async_copy(
                g_hbm.at[src_v.at[j]], rows_v.at[b], gsem.at[b]).wait()
            pltpu.async_copy(rows_v.at[b], a_sh.at[dst_v.at[j]], ssem.at[b],
                             add=True)
            jn = j + NBUF // 2

            @pl.when(jn < NB)
            def _():
                @pl.when(jn >= NBUF)
                def _():
                    pltpu.make_async_copy(
                        rows_v.at[bn], a_sh.at[dst_v.at[jn - NBUF]],
                        ssem.at[bn]).wait()
                pltpu.async_copy(
                    g_hbm.at[src_v.at[jn]], rows_v.at[bn], gsem.at[bn])

        def outer(o, carry):
            for b in range(NBUF):
                j = o * NBUF + b
                step(j, b, (b + NBUF // 2) % NBUF)
            return carry
        lax.fori_loop(0, NB // NBUF, outer, 0)

        for b in range(NBUF):
            j_last = NB - NBUF + b
            pltpu.make_async_copy(
                rows_v.at[b], a_sh.at[dst_v.at[j_last]], ssem.at[b]).wait()
    plsc.subcore_barrier()

    @pl.when(s < 10)
    def _():
        for t, a_sh in enumerate((an_sh, ar_sh)):
            pltpu.sync_copy(a_sh.at[pl.ds(r0, 1000)],
                            sp_hbm.at[c, t, pl.ds(r0, 1000)])


_agg_call = pl.kernel(
    _agg_body,
    out_type=jax.ShapeDtypeStruct((NC, 2, N, H), jnp.float32),
    mesh=_mesh,
    compiler_params=pltpu.CompilerParams(use_tc_tiling_on_sc=False),
    scratch_types=[
        pltpu.VMEM_SHARED((N, H), jnp.float32),
        pltpu.VMEM_SHARED((N, H), jnp.float32),
        pltpu.VMEM((NB, IB), jnp.int32),
        pltpu.VMEM((NB, IB), jnp.int32),
        pltpu.VMEM((NBUF, IB, H), jnp.float32),
        pltpu.SemaphoreType.DMA((NBUF,)),
        pltpu.SemaphoreType.DMA((NBUF,)),
    ],
)


def _mm_body(x_ref, w2_ref, hist_ref, gn_ref, gr_ref):
    deg = 1.0 + jnp.sum(hist_ref[...], axis=2)[:, 0, :]
    dinv = lax.rsqrt(deg)
    h = jnp.dot(x_ref[...], w2_ref[...], preferred_element_type=jnp.float32)
    gn_ref[...] = h[:, :H] * dinv[0][:, None]
    gr_ref[...] = h[:, H:] * dinv[1][:, None]


def _mm_call(x, w2, hist):
    return pl.pallas_call(
        _mm_body,
        grid=(GRID,),
        in_specs=[
            pl.BlockSpec((R, D), lambda i: (i, 0)),
            pl.BlockSpec((D, 2 * H), lambda i: (0, 0)),
            pl.BlockSpec((2, 1, NW, R), lambda i: (0, i, 0, 0)),
        ],
        out_specs=[
            pl.BlockSpec((R, H), lambda i: (i, 0)),
            pl.BlockSpec((R, H), lambda i: (i, 0)),
        ],
        out_shape=[
            jax.ShapeDtypeStruct((N, H), jnp.float32),
            jax.ShapeDtypeStruct((N, H), jnp.float32),
        ],
    )(x, w2, hist)


def _fin_body(sp_ref, gn_ref, gr_ref, hist_ref, bb_ref, lw_ref, lb_ref, out_ref):
    deg = 1.0 + jnp.sum(hist_ref[...], axis=2)[:, 0, :]
    dinv = lax.rsqrt(deg)
    sp = sp_ref[...]
    yn = (sp[0, 0] + sp[1, 0] - gn_ref[...]) * dinv[0][:, None]
    yr = (sp[0, 1] + sp[1, 1] - gr_ref[...]) * dinv[1][:, None]
    y = yn + yr + (bb_ref[0] + bb_ref[1])[None, :]
    r = jnp.maximum(y, 0.0)
    out_ref[...] = (
        jnp.dot(r, lw_ref[...], preferred_element_type=jnp.float32) + lb_ref[0, 0]
    )


def _fin_call(sp, gn, gr, hist, bb, lw, lb):
    return pl.pallas_call(
        _fin_body,
        grid=(GRID,),
        in_specs=[
            pl.BlockSpec((NC, 2, R, H), lambda i: (0, 0, i, 0)),
            pl.BlockSpec((R, H), lambda i: (i, 0)),
            pl.BlockSpec((R, H), lambda i: (i, 0)),
            pl.BlockSpec((2, 1, NW, R), lambda i: (0, i, 0, 0)),
            pl.BlockSpec((2, H), lambda i: (0, 0)),
            pl.BlockSpec((H, 1), lambda i: (0, 0)),
            pl.BlockSpec((1, 1), lambda i: (0, 0)),
        ],
        out_specs=pl.BlockSpec((R, 1), lambda i: (i, 0)),
        out_shape=jax.ShapeDtypeStruct((N, 1), jnp.float32),
    )(sp, gn, gr, hist, bb, lw, lb)


def kernel(x_house, edge_index_near, edge_index_road,
           W_near, b_near, W_road, b_road, lin_W, lin_b):
    dst2 = jnp.stack([edge_index_near[1], edge_index_road[1]]).reshape(2, NW, EW)
    hist = _deg_call(dst2)
    w2 = jnp.concatenate([W_near, W_road], axis=1)
    gn, gr = _mm_call(x_house, w2, hist)
    src4 = jnp.stack([edge_index_near[0], edge_index_road[0]]).reshape(2, NW, NB, IB)
    dst4 = dst2.reshape(2, NW, NB, IB)
    sp = _agg_call(gn, gr, src4, dst4)
    bb = jnp.stack([b_near, b_road])
    out = _fin_call(sp, gn, gr, hist, bb, lin_W, lin_b.reshape(1, 1))
    return out[:, 0]
